# baseline (speedup 1.0000x reference)
import functools

import jax
import jax.numpy as jnp
from jax import lax
from jax.experimental import pallas as pl
from jax.experimental.pallas import tpu as pltpu
from jax.experimental.pallas import tpu_sc as plsc

B = 8
N = 4096
M = 1024
CIN = 128
SKIP = 128
COUT = 128
K = 3
P = B * N

TN = 512
NT = N // TN

NW = 32
CHUNK = P // NW
G = 64
NG = CHUNK // G


def _knn_body(pts_ref, cent_ref, gidx_ref, wn_ref):
    b = pl.program_id(0)
    pts = pts_ref[0]
    cent = cent_ref[0]
    work = jnp.zeros((TN, M), jnp.float32)
    for d in range(3):
        diff = pts[:, d:d + 1] - cent[d:d + 1, :]
        work = work + diff * diff
    col = lax.broadcasted_iota(jnp.int32, (TN, M), 1)
    idxs, vals = [], []
    for _ in range(K):
        mval = jnp.min(work, axis=1, keepdims=True)
        cand = jnp.where(work == mval, col, M)
        ik = jnp.min(cand, axis=1, keepdims=True)
        idxs.append(ik)
        vals.append(mval)
        work = jnp.where(col == ik, jnp.float32(jnp.inf), work)
    ws = [1.0 / jnp.maximum(v, 1e-16) for v in vals]
    wt = ws[0] + ws[1] + ws[2]
    zi = jnp.zeros((TN, 8 - K), jnp.int32)
    zf = jnp.zeros((TN, 8 - K), jnp.float32)
    gidx_ref[0] = jnp.concatenate([i + b * M for i in idxs] + [zi], axis=1)
    wn_ref[0] = jnp.concatenate([w / wt for w in ws] + [zf], axis=1)


def _knn(ptsT8, cent8):
    return pl.pallas_call(
        _knn_body,
        grid=(B, NT),
        in_specs=[
            pl.BlockSpec((1, TN, 8), lambda b, t: (b, t, 0)),
            pl.BlockSpec((1, 8, M), lambda b, t: (b, 0, 0)),
        ],
        out_specs=[
            pl.BlockSpec((1, TN, 8), lambda b, t: (b, t, 0)),
            pl.BlockSpec((1, TN, 8), lambda b, t: (b, t, 0)),
        ],
        out_shape=[
            jax.ShapeDtypeStruct((B, N, 8), jnp.int32),
            jax.ShapeDtypeStruct((B, N, 8), jnp.float32),
        ],
    )(ptsT8, cent8)


def _sc_interp_body(cf_hbm, g0_hbm, g1_hbm, g2_hbm, w0_hbm, w1_hbm, w2_hbm,
                    out_hbm,
                    i0, i1, i2,
                    a0, a1, a2, aw0, aw1, aw2,
                    b0, b1, b2, bw0, bw1, bw2, ob, sa, sb):
    wid = lax.axis_index("s") * 2 + lax.axis_index("c")
    pltpu.sync_copy(g0_hbm.at[wid], i0)
    pltpu.sync_copy(g1_hbm.at[wid], i1)
    pltpu.sync_copy(g2_hbm.at[wid], i2)
    base = wid * CHUNK
    bufs = [(a0, a1, a2, aw0, aw1, aw2, sa), (b0, b1, b2, bw0, bw1, bw2, sb)]

    def start(g):
        t0, t1, t2, u0, u1, u2, sem = bufs[g % 2]
        wrow = wid * NG + g
        return (pltpu.async_copy(cf_hbm.at[i0.at[g]], t0, sem),
                pltpu.async_copy(cf_hbm.at[i1.at[g]], t1, sem),
                pltpu.async_copy(cf_hbm.at[i2.at[g]], t2, sem),
                pltpu.async_copy(w0_hbm.at[wrow], u0, sem),
                pltpu.async_copy(w1_hbm.at[wrow], u1, sem),
                pltpu.async_copy(w2_hbm.at[wrow], u2, sem))

    pend = start(0)
    for g in range(NG):
        for c in pend:
            c.wait()
        t0, t1, t2, u0, u1, u2, _ = bufs[g % 2]
        if g + 1 < NG:
            pend = start(g + 1)

        def pbody(p, carry, t0=t0, t1=t1, t2=t2, u0=u0, u1=u1, u2=u2):
            w0 = u0[p]
            w1 = u1[p]
            w2 = u2[p]
            for c in range(CIN // 16):
                sl = pl.ds(c * 16, 16)
                ob[p, sl] = w0 * t0[p, sl] + w1 * t1[p, sl] + w2 * t2[p, sl]
            return carry

        lax.fori_loop(0, G, pbody, 0)
        pltpu.sync_copy(ob, out_hbm.at[pl.ds(base + g * G, G)])


def _sc_interp(cf_flat, g0, g1, g2, w0b, w1b, w2b):
    f32 = jnp.float32
    call = pl.kernel(
        _sc_interp_body,
        out_type=jax.ShapeDtypeStruct((P, CIN), f32),
        mesh=plsc.VectorSubcoreMesh(core_axis_name="c", subcore_axis_name="s"),
        scratch_types=[
            pltpu.VMEM((NG, G), jnp.int32),
            pltpu.VMEM((NG, G), jnp.int32),
            pltpu.VMEM((NG, G), jnp.int32),
            pltpu.VMEM((G, CIN), f32),
            pltpu.VMEM((G, CIN), f32),
            pltpu.VMEM((G, CIN), f32),
            pltpu.VMEM((G, 16), f32),
            pltpu.VMEM((G, 16), f32),
            pltpu.VMEM((G, 16), f32),
            pltpu.VMEM((G, CIN), f32),
            pltpu.VMEM((G, CIN), f32),
            pltpu.VMEM((G, CIN), f32),
            pltpu.VMEM((G, 16), f32),
            pltpu.VMEM((G, 16), f32),
            pltpu.VMEM((G, 16), f32),
            pltpu.VMEM((G, CIN), f32),
            pltpu.SemaphoreType.DMA,
            pltpu.SemaphoreType.DMA,
        ],
    )
    return call(cf_flat, g0, g1, g2, w0b, w1b, w2b)


def _mlp1_body(pf_ref, it_ref, W1a_ref, W1b_ref, b1_ref, h_ref, st_ref):
    hT = lax.dot_general(pf_ref[0], W1a_ref[...], (((0,), (1,)), ((), ())),
                         preferred_element_type=jnp.float32)
    hT = hT + lax.dot_general(it_ref[...], W1b_ref[...],
                              (((1,), (1,)), ((), ())),
                              preferred_element_type=jnp.float32)
    hT = hT + b1_ref[...]
    h_ref[...] = hT
    s = jnp.sum(hT, axis=0, keepdims=True)
    q = jnp.sum(hT * hT, axis=0, keepdims=True)
    ri = lax.broadcasted_iota(jnp.int32, (8, COUT), 0)
    upd = (jnp.where(ri == 0, jnp.broadcast_to(s, (8, COUT)), 0.0) +
           jnp.where(ri == 1, jnp.broadcast_to(q, (8, COUT)), 0.0))
    first = jnp.logical_and(pl.program_id(0) == 0, pl.program_id(1) == 0)

    @pl.when(first)
    def _():
        st_ref[...] = upd

    @pl.when(jnp.logical_not(first))
    def _():
        st_ref[...] = st_ref[...] + upd


def _mlp1(pf, interp, W1a, W1b, b1row):
    return pl.pallas_call(
        _mlp1_body,
        grid=(B, NT),
        in_specs=[
            pl.BlockSpec((1, SKIP, TN), lambda b, t: (b, 0, t)),
            pl.BlockSpec((TN, CIN), lambda b, t: (b * NT + t, 0)),
            pl.BlockSpec((COUT, SKIP), lambda b, t: (0, 0)),
            pl.BlockSpec((COUT, CIN), lambda b, t: (0, 0)),
            pl.BlockSpec((1, COUT), lambda b, t: (0, 0)),
        ],
        out_specs=[
            pl.BlockSpec((TN, COUT), lambda b, t: (b * NT + t, 0)),
            pl.BlockSpec((8, COUT), lambda b, t: (0, 0)),
        ],
        out_shape=[
            jax.ShapeDtypeStruct((P, COUT), jnp.float32),
            jax.ShapeDtypeStruct((8, COUT), jnp.float32),
        ],
    )(pf, interp, W1a, W1b, b1row)


def _mlp2_body(h_ref, st_ref, gam_ref, bet_ref, W2_ref, b2_ref, out_ref):
    inv_p = jnp.float32(1.0 / P)
    mean = st_ref[0:1, :] * inv_p
    ex2 = st_ref[1:2, :] * inv_p
    var = ex2 - mean * mean
    rstd = lax.rsqrt(var + 1e-5)
    scale = gam_ref[...] * rstd
    shift = bet_ref[...] - mean * scale
    hr = jnp.maximum(h_ref[...] * scale + shift, 0.0)
    o = lax.dot_general(W2_ref[...], hr, (((1,), (1,)), ((), ())),
                        preferred_element_type=jnp.float32)
    out_ref[0] = o + b2_ref[...]


def _mlp2(h, st, gamma_row, beta_row, W2, b2col):
    return pl.pallas_call(
        _mlp2_body,
        grid=(B, NT),
        in_specs=[
            pl.BlockSpec((TN, COUT), lambda b, t: (b * NT + t, 0)),
            pl.BlockSpec((8, COUT), lambda b, t: (0, 0)),
            pl.BlockSpec((1, COUT), lambda b, t: (0, 0)),
            pl.BlockSpec((1, COUT), lambda b, t: (0, 0)),
            pl.BlockSpec((COUT, COUT), lambda b, t: (0, 0)),
            pl.BlockSpec((COUT, 1), lambda b, t: (0, 0)),
        ],
        out_specs=pl.BlockSpec((1, COUT, TN), lambda b, t: (b, 0, t)),
        out_shape=jax.ShapeDtypeStruct((B, COUT, N), jnp.float32),
    )(h, st, gamma_row, beta_row, W2, b2col)


def kernel(points, point_features, centroids, centroid_features,
           W1, b1, gamma, beta, W2, b2):
    f32 = jnp.float32
    ptsT8 = jnp.pad(jnp.transpose(points, (0, 2, 1)), ((0, 0), (0, 0), (0, 5)))
    cent8 = jnp.pad(centroids, ((0, 0), (0, 5), (0, 0)))
    gidx, wn = _knn(ptsT8, cent8)

    cf_flat = jnp.transpose(centroid_features, (0, 2, 1)).reshape(B * M, CIN)
    g0 = gidx[:, :, 0].reshape(NW, NG, G)
    g1 = gidx[:, :, 1].reshape(NW, NG, G)
    g2 = gidx[:, :, 2].reshape(NW, NG, G)
    wb = [jnp.broadcast_to(wn[:, :, k].reshape(P, 1),
                           (P, 16)).reshape(NW * NG, G, 16) for k in range(K)]
    interp = _sc_interp(cf_flat, g0, g1, g2, wb[0], wb[1], wb[2])

    W1a = W1[:, :SKIP]
    W1b = W1[:, SKIP:]
    h, st = _mlp1(point_features, interp, W1a, W1b,
                  b1.reshape(1, COUT).astype(f32))
    out = _mlp2(h, st, gamma.reshape(1, COUT), beta.reshape(1, COUT),
                W2, b2.reshape(COUT, 1))
    return out
